# Initial kernel scaffold; baseline (speedup 1.0000x reference)
import jax
import jax.numpy as jnp
from jax.experimental import pallas as pl


def kernel(x):
    raise NotImplementedError("write your pallas kernel here")


# speedup vs baseline: 1.6974x; 1.6974x over previous
import functools

import jax
import jax.numpy as jnp
from jax import lax
from jax.experimental import pallas as pl
from jax.experimental.pallas import tpu as pltpu
from jax.experimental.pallas import tpu_sc as plsc

_N = 2048
_K = 8192
_D = 128
_BQ = 128
_BK = 256


def _l1_argmin_body(a_ref, gt_ref, idx_ref, minv_ref, mini_ref):
    j = pl.program_id(1)
    nk = pl.num_programs(1)
    a = a_ref[...]
    acc = jnp.zeros((_BQ, _BK), jnp.float32)
    for d in range(_D):
        acc = acc + jnp.abs(a[:, d:d + 1] - gt_ref[d:d + 1, :])
    bmin = jnp.min(acc, axis=1, keepdims=True)
    ii = lax.broadcasted_iota(jnp.int32, (_BQ, _BK), 1)
    barg = jnp.min(jnp.where(acc == bmin, ii, _K), axis=1,
                   keepdims=True) + j * _BK

    @pl.when(j == 0)
    def _():
        minv_ref[...] = bmin
        mini_ref[...] = barg

    @pl.when(j > 0)
    def _():
        upd = bmin < minv_ref[...]
        mini_ref[...] = jnp.where(upd, barg, mini_ref[...])
        minv_ref[...] = jnp.where(upd, bmin, minv_ref[...])

    @pl.when(j == nk - 1)
    def _():
        idx_ref[0, :, :] = mini_ref[...]


def _l1_argmin(agents, goals_t):
    grid = (_N // _BQ, _K // _BK)
    idx3 = pl.pallas_call(
        _l1_argmin_body,
        grid=grid,
        in_specs=[
            pl.BlockSpec((_BQ, _D), lambda i, j: (i, 0)),
            pl.BlockSpec((_D, _BK), lambda i, j: (0, j)),
        ],
        out_specs=pl.BlockSpec((1, _BQ, 1), lambda i, j: (i, 0, 0)),
        out_shape=jax.ShapeDtypeStruct((_N // _BQ, _BQ, 1), jnp.int32),
        scratch_shapes=[
            pltpu.VMEM((_BQ, 1), jnp.float32),
            pltpu.VMEM((_BQ, 1), jnp.int32),
        ],
        compiler_params=pltpu.CompilerParams(
            dimension_semantics=("parallel", "arbitrary")),
    )(agents, goals_t)
    return idx3.reshape(_N)


_NC = 2
_NS = 16
_NW = _NC * _NS
_BPW = _N // _NW


def _sc_gather_sub_body(goals_hbm, idx_hbm, agents_hbm, out_hbm,
                        idx_v, g_v, a_v, sem):
    wid = lax.axis_index("s") * _NC + lax.axis_index("c")
    base = wid * _BPW
    pltpu.sync_copy(idx_hbm.at[pl.ds(base, _BPW)], idx_v)
    cp = pltpu.async_copy(goals_hbm.at[idx_v], g_v, sem)
    pltpu.sync_copy(agents_hbm.at[pl.ds(base, _BPW)], a_v)
    cp.wait()

    def row(i, c):
        for jj in range(_D // 16):
            s = slice(jj * 16, (jj + 1) * 16)
            g_v[i, s] = g_v[i, s] - a_v[i, s]
        return c

    lax.fori_loop(0, _BPW, row, 0)
    pltpu.sync_copy(g_v, out_hbm.at[pl.ds(base, _BPW)])


def _sc_gather_sub(goals, idx, agents):
    mesh = plsc.VectorSubcoreMesh(core_axis_name="c", subcore_axis_name="s")
    k = functools.partial(
        pl.kernel,
        mesh=mesh,
        out_type=jax.ShapeDtypeStruct((_N, _D), jnp.float32),
        scratch_types=[
            pltpu.VMEM((_BPW,), jnp.int32),
            pltpu.VMEM((_BPW, _D), jnp.float32),
            pltpu.VMEM((_BPW, _D), jnp.float32),
            pltpu.SemaphoreType.DMA,
        ],
    )(_sc_gather_sub_body)
    return k(goals, idx, agents)


def kernel(x):
    agents = x[:_N, :_D]
    goals = x[_N:, :_D]
    goals_t = goals.T
    idx = _l1_argmin(agents, goals_t)
    dis = _sc_gather_sub(goals, idx, agents)
    return idx, dis
